# baseline (speedup 1.0000x reference)
import functools

import jax
import jax.numpy as jnp
from jax import lax
from jax.experimental import pallas as pl
from jax.experimental.pallas import tpu as pltpu
from jax.experimental.pallas import tpu_sc as plsc

_N = 50000
_E = 800000
_D = 32
_R = 4
_H1 = 32
_H2 = 16

_NC, _NS = 2, 16
_NW = _NC * _NS
_PER_W = _E // _NW
_CHUNK = 1000
_NITER = _PER_W // _CHUNK

_B = 3200
_NEG = -1.0e30


def _sc_gather(x, idx2):
    mesh = plsc.VectorSubcoreMesh(core_axis_name="c", subcore_axis_name="s")

    @functools.partial(
        pl.kernel,
        mesh=mesh,
        compiler_params=pltpu.CompilerParams(use_tc_tiling_on_sc=False),
        out_type=jax.ShapeDtypeStruct((_E, 128), jnp.float32),
        scratch_types=[
            pltpu.VMEM((_CHUNK,), jnp.int32),
            pltpu.VMEM((_CHUNK,), jnp.int32),
            pltpu.VMEM((_CHUNK, _D), jnp.float32),
            pltpu.VMEM((_CHUNK, _D), jnp.float32),
            pltpu.SemaphoreType.DMA,
            pltpu.SemaphoreType.DMA,
        ],
    )
    def gather_k(x_hbm, idx2_hbm, out_hbm,
                 idxr_v, idxc_v, rowsr_v, rowsc_v, semr, semc):
        wid = lax.axis_index("s") * _NC + lax.axis_index("c")
        base = wid * _PER_W

        def body(i, carry):
            off = base + i * _CHUNK
            pltpu.sync_copy(idx2_hbm.at[0, pl.ds(off, _CHUNK)], idxr_v)
            pltpu.sync_copy(idx2_hbm.at[1, pl.ds(off, _CHUNK)], idxc_v)
            cr = pltpu.async_copy(x_hbm.at[idxr_v], rowsr_v, semr)
            cc = pltpu.async_copy(x_hbm.at[idxc_v], rowsc_v, semc)
            cr.wait()
            cc.wait()
            pltpu.sync_copy(rowsr_v, out_hbm.at[pl.ds(off, _CHUNK), pl.ds(0, _D)])
            pltpu.sync_copy(rowsc_v, out_hbm.at[pl.ds(off, _CHUNK), pl.ds(_D, _D)])
            return carry

        lax.fori_loop(0, _NITER, body, 0)

    return gather_k(x, idx2)


def _sdot(st, v):
    return lax.dot_general(st, v, (((0,), (0,)), ((), ())),
                           preferred_element_type=jnp.float32)


def _tc_body(z_ref, s_ref, w_ref, v8_ref, w2t_ref, b2_ref, w3_ref, b3_ref,
             o_ref):
    z = z_ref[...]
    st = s_ref[...]
    zlane = lax.broadcasted_iota(jnp.int32, (_B, 128), 1)
    zs = jnp.where(zlane < 2 * _D, z, 0.0)
    h = (jnp.dot(zs, w_ref[...], preferred_element_type=jnp.float32)
         + _sdot(st, v8_ref[...]))
    hm = jnp.maximum(h, 0.0)
    h2 = jnp.dot(hm, w2t_ref[...], preferred_element_type=jnp.float32)
    h2 = jnp.maximum(h2 + b2_ref[...], 0.0)
    o = lax.dot_general(w3_ref[...], h2, (((0,), (1,)), ((), ())),
                        preferred_element_type=jnp.float32)
    o_ref[...] = o + b3_ref[...]


def _tc_mlp(z, s, w128, v8, w2t, b2, W3, b3):
    grid = (_E // _B,)
    full = lambda sh: pl.BlockSpec(sh, lambda i: (0, 0))
    return pl.pallas_call(
        _tc_body,
        grid=grid,
        in_specs=[
            pl.BlockSpec((_B, 128), lambda i: (i, 0)),
            pl.BlockSpec((8, _B), lambda i: (0, i)),
            full((128, _R * _H1)),
            full((8, _R * _H1)),
            full((_R * _H1, _H2)),
            full((1, _H2)),
            full((_H2, 1)),
            full((1, 1)),
        ],
        out_specs=pl.BlockSpec((1, _B), lambda i: (0, i)),
        out_shape=jax.ShapeDtypeStruct((1, _E), jnp.float32),
    )(z, s, w128, v8, w2t, b2, W3, b3)


def kernel(edge_index, relations, concs, x, W1, b1, W2, b2, W3, b3):
    with jax.ensure_compile_time_eval():
        maskb = jax.random.uniform(jax.random.key(42), (_E,)) >= 0.5

    t = edge_index.T.astype(jnp.int32)
    idx2 = jnp.where(maskb[None, :], t, t[::-1])
    z = _sc_gather(x, idx2)

    ct = concs.T
    c2 = jnp.where(maskb[None, :], ct, ct[::-1])
    oh = (relations[None, :] == jnp.arange(_R)[:, None]).astype(jnp.float32)
    onesr = jnp.ones((1, _E), jnp.float32)
    st = jnp.concatenate([c2, oh, onesr, jnp.zeros((1, _E), jnp.float32)],
                         axis=0)

    kdim = _R * _H1
    w1cat = W1.transpose(1, 0, 2).reshape(2 * (_D + 1), kdim)
    w1a = w1cat[0:_D]
    wca = w1cat[_D:_D + 1]
    w1b = w1cat[_D + 1:2 * _D + 1]
    wcb = w1cat[2 * _D + 1:2 * _D + 2]

    w128 = jnp.concatenate(
        [w1a, w1b, jnp.zeros((128 - 2 * _D, kdim), jnp.float32)], axis=0)
    lanegrp = jnp.arange(kdim)[None, :] // _H1
    vmask = jnp.where(lanegrp == jnp.arange(_R)[:, None], 0.0, _NEG)
    v8 = jnp.concatenate(
        [wca, wcb, vmask, b1.reshape(1, kdim),
         jnp.zeros((1, kdim), jnp.float32)], axis=0)
    w2t = jnp.tile(W2, (_R, 1))

    out = _tc_mlp(z, st, w128, v8, w2t, b2.reshape(1, _H2), W3,
                  b3.reshape(1, 1))
    return out.reshape(_E, 1)


# speedup vs baseline: 8.5261x; 8.5261x over previous
import functools

import jax
import jax.numpy as jnp
from jax import lax
from jax.experimental import pallas as pl
from jax.experimental.pallas import tpu as pltpu
from jax.experimental.pallas import tpu_sc as plsc

_N = 50000
_E = 800000
_D = 32
_R = 4
_H1 = 32
_H2 = 16

_NC, _NS = 2, 16
_NW = _NC * _NS
_PER_W = _E // _NW
_CHUNK = 1000
_NITER = _PER_W // _CHUNK

_B = 3200
_NEG = -1.0e30


def _sc_gather(x, idx2):
    mesh = plsc.VectorSubcoreMesh(core_axis_name="c", subcore_axis_name="s")

    @functools.partial(
        pl.kernel,
        mesh=mesh,
        compiler_params=pltpu.CompilerParams(use_tc_tiling_on_sc=False),
        out_type=jax.ShapeDtypeStruct((_E, 128), jnp.float32),
        scratch_types=[
            pltpu.VMEM((_CHUNK,), jnp.int32),
            pltpu.VMEM((_CHUNK,), jnp.int32),
            pltpu.VMEM((_CHUNK, _D), jnp.float32),
            pltpu.VMEM((_CHUNK, _D), jnp.float32),
            pltpu.SemaphoreType.DMA,
            pltpu.SemaphoreType.DMA,
        ],
    )
    def gather_k(x_hbm, idx2_hbm, out_hbm,
                 idxr_v, idxc_v, rowsr_v, rowsc_v, semr, semc):
        wid = lax.axis_index("s") * _NC + lax.axis_index("c")
        base = wid * _PER_W

        def body(i, carry):
            off = base + i * _CHUNK
            pltpu.sync_copy(idx2_hbm.at[0, pl.ds(off, _CHUNK)], idxr_v)
            pltpu.sync_copy(idx2_hbm.at[1, pl.ds(off, _CHUNK)], idxc_v)
            cr = pltpu.async_copy(x_hbm.at[idxr_v], rowsr_v, semr)
            cc = pltpu.async_copy(x_hbm.at[idxc_v], rowsc_v, semc)
            cr.wait()
            cc.wait()
            pltpu.sync_copy(rowsr_v, out_hbm.at[pl.ds(off, _CHUNK), pl.ds(0, _D)])
            pltpu.sync_copy(rowsc_v, out_hbm.at[pl.ds(off, _CHUNK), pl.ds(_D, _D)])
            return carry

        lax.fori_loop(0, _NITER, body, 0)

    return gather_k(x, idx2)


def _sdot(st, v):
    return lax.dot_general(st, v, (((0,), (0,)), ((), ())),
                           preferred_element_type=jnp.float32)


def _tc_body(z_ref, s_ref, w_ref, v8_ref, w2t_ref, b2_ref, w3_ref, b3_ref,
             o_ref):
    z = z_ref[...]
    st = s_ref[...]
    zlane = lax.broadcasted_iota(jnp.int32, (_B, 128), 1)
    zs = jnp.where(zlane < 2 * _D, z, 0.0)
    h = (jnp.dot(zs, w_ref[...], preferred_element_type=jnp.float32)
         + _sdot(st, v8_ref[...]))
    hm = jnp.maximum(h, 0.0)
    h2 = jnp.dot(hm, w2t_ref[...], preferred_element_type=jnp.float32)
    h2 = jnp.maximum(h2 + b2_ref[...], 0.0)
    o = lax.dot_general(w3_ref[...], h2, (((0,), (1,)), ((), ())),
                        preferred_element_type=jnp.float32)
    o_ref[...] = o + b3_ref[...]


def _tc_mlp(z, s, w128, v8, w2t, b2, W3, b3):
    grid = (_E // _B,)
    full = lambda sh: pl.BlockSpec(sh, lambda i: (0, 0))
    return pl.pallas_call(
        _tc_body,
        grid=grid,
        in_specs=[
            pl.BlockSpec((_B, 128), lambda i: (i, 0)),
            pl.BlockSpec((8, _B), lambda i: (0, i)),
            full((128, _R * _H1)),
            full((8, _R * _H1)),
            full((_R * _H1, _H2)),
            full((1, _H2)),
            full((_H2, 1)),
            full((1, 1)),
        ],
        out_specs=pl.BlockSpec((1, _B), lambda i: (0, i)),
        out_shape=jax.ShapeDtypeStruct((1, _E), jnp.float32),
    )(z, s, w128, v8, w2t, b2, W3, b3)


def kernel(edge_index, relations, concs, x, W1, b1, W2, b2, W3, b3):
    with jax.ensure_compile_time_eval():
        maskb = jax.random.uniform(jax.random.key(42), (_E,)) >= 0.5

    idxr = edge_index[:, 0].astype(jnp.int32)
    idxc = edge_index[:, 1].astype(jnp.int32)
    idx2 = jnp.stack([jnp.where(maskb, idxr, idxc),
                      jnp.where(maskb, idxc, idxr)], axis=0)
    z = _sc_gather(x, idx2)

    c0, c1 = concs[:, 0], concs[:, 1]
    ca = jnp.where(maskb, c0, c1)[None, :]
    cb = jnp.where(maskb, c1, c0)[None, :]
    oh = (relations[None, :] == jnp.arange(_R)[:, None]).astype(jnp.float32)
    onesr = jnp.ones((1, _E), jnp.float32)
    st = jnp.concatenate([ca, cb, oh, onesr, jnp.zeros((1, _E), jnp.float32)],
                         axis=0)

    kdim = _R * _H1
    w1cat = W1.transpose(1, 0, 2).reshape(2 * (_D + 1), kdim)
    w1a = w1cat[0:_D]
    wca = w1cat[_D:_D + 1]
    w1b = w1cat[_D + 1:2 * _D + 1]
    wcb = w1cat[2 * _D + 1:2 * _D + 2]

    w128 = jnp.concatenate(
        [w1a, w1b, jnp.zeros((128 - 2 * _D, kdim), jnp.float32)], axis=0)
    lanegrp = jnp.arange(kdim)[None, :] // _H1
    vmask = jnp.where(lanegrp == jnp.arange(_R)[:, None], 0.0, _NEG)
    v8 = jnp.concatenate(
        [wca, wcb, vmask, b1.reshape(1, kdim),
         jnp.zeros((1, kdim), jnp.float32)], axis=0)
    w2t = jnp.tile(W2, (_R, 1))

    out = _tc_mlp(z, st, w128, v8, w2t, b2.reshape(1, _H2), W3,
                  b3.reshape(1, 1))
    return out.reshape(_E, 1)
